# baseline (speedup 1.0000x reference)
import functools

import jax
import jax.numpy as jnp
from jax import lax
from jax.experimental import pallas as pl
from jax.experimental.pallas import tpu as pltpu
from jax.experimental.pallas import tpu_sc as plsc

N_EMB = 1000000
D_EMB = 64
BATCH = 16384

_info = plsc.get_sparse_core_info()
_NC, _NS = _info.num_cores, _info.num_subcores
_NW = _NC * _NS
_SHARD = 31232
_CHUNK = 512
_NCHUNK = _SHARD // _CHUNK
_DUMP = BATCH
_SCR_ROWS = BATCH + 8

_mesh = plsc.VectorSubcoreMesh(core_axis_name="c", subcore_axis_name="s")
_params = pltpu.CompilerParams(needs_layout_passes=False)


@functools.partial(
    pl.kernel,
    mesh=_mesh,
    out_type=jax.ShapeDtypeStruct((_SCR_ROWS, 128), jnp.float32),
    compiler_params=_params,
    scratch_types=[
        pltpu.VMEM((BATCH,), jnp.int32),
        pltpu.VMEM((BATCH,), jnp.int32),
        pltpu.VMEM((BATCH,), jnp.int32),
        pltpu.VMEM((BATCH,), jnp.int32),
        pltpu.VMEM((D_EMB, _CHUNK), jnp.float32),
        pltpu.VMEM((2, 64, 128), jnp.float32),
        pltpu.VMEM((2, 64), jnp.int32),
        pltpu.SemaphoreType.DMA,
        pltpu.SemaphoreType.DMA,
    ],
)
def _scan_gather(x_hbm, embt_hbm, tail_hbm, out_hbm, idx_v, mb_v,
                 cb_v, cr_v, stage_v, acc_v, aidx_v, sem, sem_sc):
    wid = lax.axis_index("s") * _NC + lax.axis_index("c")
    lo = wid * _SHARD
    hi = jnp.where(wid == _NW - 1, N_EMB, lo + _SHARD).astype(jnp.int32)
    iota = lax.iota(jnp.int32, 16)

    pltpu.sync_copy(x_hbm, idx_v)

    def prefilter(g, off):
        v = idx_v[pl.ds(g * 16, 16)]
        m = (v >= lo) & (v < hi)
        plsc.store_compressed(mb_v.at[pl.ds(off, 16)], g * 16 + iota, mask=m)
        return off + plsc.all_reduce_population_count(m)[0]

    n_match = lax.fori_loop(0, BATCH // 16, prefilter, jnp.int32(0))
    n_groups = (n_match + 15) // 16

    def drain_one():
        pltpu.make_async_copy(
            embt_hbm.at[pl.ds(0, 64), pl.ds(0, 128)], acc_v.at[0],
            sem_sc).wait()

    def process_chunk(rlo, carry):
        rhi = rlo + _CHUNK

        def rescan(g, off):
            mg = (g * 16 + iota) < n_match
            bv = jnp.where(mg, mb_v[pl.ds(g * 16, 16)], 0)
            rv = plsc.load_gather(idx_v, [bv])
            m = mg & (rv >= rlo) & (rv < rhi)
            plsc.store_compressed(cb_v.at[pl.ds(off, 16)], bv, mask=m)
            plsc.store_compressed(cr_v.at[pl.ds(off, 16)], rv, mask=m)
            return off + plsc.all_reduce_population_count(m)[0]

        n2 = lax.fori_loop(0, n_groups, rescan, jnp.int32(0))

        def extract(h, ec):
            fill, slot, o = ec
            fresh = (fill == 0) & (o >= 2)

            @pl.when(fresh)
            def _():
                drain_one()

            o = jnp.where(fresh, o - 1, o)
            b16 = cb_v[pl.ds(h * 16, 16)]
            r16 = cr_v[pl.ds(h * 16, 16)] - rlo
            mk = (h * 16 + iota) < n2
            r16 = jnp.where(mk, r16, 0)
            slot16 = jnp.full((16,), 0, jnp.int32) + slot
            f16 = fill + iota

            def move_d(dd, _):
                dsplat = jnp.full((16,), 0, jnp.int32) + dd
                val = plsc.load_gather(stage_v, [dsplat, r16], mask=mk)
                plsc.store_scatter(acc_v, [slot16, f16, dsplat], val, mask=mk)
                return 0

            lax.fori_loop(0, D_EMB, move_d, 0)
            plsc.store_scatter(aidx_v, [slot16, f16],
                               jnp.where(mk, b16, _DUMP))
            fill = fill + 16
            fl = fill == 64

            @pl.when(fl)
            def _():
                pltpu.async_copy(acc_v.at[slot], out_hbm.at[aidx_v.at[slot]],
                                 sem_sc)

            return (jnp.where(fl, 0, fill), jnp.where(fl, 1 - slot, slot),
                    o + fl.astype(jnp.int32))

        return lax.fori_loop(0, (n2 + 15) // 16, extract, carry)

    def stage_chunk(rlo):
        cps = [
            pltpu.async_copy(
                embt_hbm.at[pl.ds(i * 8, 8),
                            pl.ds(pl.multiple_of(rlo, 128), _CHUNK)],
                stage_v.at[pl.ds(i * 8, 8), pl.ds(0, _CHUNK)],
                sem)
            for i in range(8)
        ]
        for cp in cps:
            cp.wait()

    def chunk_body(c, carry):
        rlo = lo + c * _CHUNK

        @pl.when(c < _NCHUNK + 1)
        def _():
            stage_chunk(rlo)

        @pl.when(c == _NCHUNK + 1)
        def _():
            pltpu.sync_copy(tail_hbm, stage_v.at[:, pl.ds(0, 128)])

        return process_chunk(rlo, carry)

    n_chunks = jnp.where(wid == _NW - 1, _NCHUNK + 2, _NCHUNK)
    fill, slot, o = lax.fori_loop(
        0, n_chunks, chunk_body,
        (jnp.int32(0), jnp.int32(0), jnp.int32(0)))

    fresh = (fill > 0) & (o >= 2)

    @pl.when(fresh)
    def _():
        drain_one()

    o = jnp.where(fresh, o - 1, o)

    @pl.when(fill > 0)
    def _():
        slot16 = jnp.full((16,), 0, jnp.int32) + slot

        def pad(k, _):
            plsc.store_scatter(aidx_v, [slot16, fill + k * 16 + iota],
                               jnp.full((16,), _DUMP, jnp.int32))
            return 0

        lax.fori_loop(0, (64 - fill) // 16, pad, 0)
        pltpu.async_copy(acc_v.at[slot], out_hbm.at[aidx_v.at[slot]], sem_sc)

    o = o + (fill > 0).astype(jnp.int32)

    def fdrain(k, _):
        drain_one()
        return 0

    lax.fori_loop(0, o, fdrain, 0)


_BPW = BATCH // _NW


@functools.partial(
    pl.kernel,
    mesh=_mesh,
    out_type=jax.ShapeDtypeStruct((D_EMB, BATCH), jnp.float32),
    compiler_params=_params,
    scratch_types=[
        pltpu.VMEM((_BPW, 128), jnp.float32),
        pltpu.VMEM((D_EMB, _BPW), jnp.float32),
        pltpu.SemaphoreType.DMA,
    ],
)
def _transpose_out(scr_hbm, out_hbm, st_v, ob_v, sem):
    wid = lax.axis_index("s") * _NC + lax.axis_index("c")
    b0 = wid * _BPW
    iota = lax.iota(jnp.int32, 16)
    pltpu.sync_copy(scr_hbm.at[pl.ds(pl.multiple_of(b0, 8), _BPW)], st_v)

    def grp(h, _):
        b16 = h * 16 + iota
        for dd in range(D_EMB):
            val = plsc.load_gather(st_v, [b16, jnp.full((16,), dd, jnp.int32)])
            ob_v[dd, pl.ds(h * 16, 16)] = val
        return 0

    lax.fori_loop(0, _BPW // 16, grp, 0)
    pltpu.sync_copy(ob_v, out_hbm.at[:, pl.ds(pl.multiple_of(b0, 128), _BPW)])


def kernel(x, emb):
    tail = jnp.zeros((D_EMB, 128), jnp.float32)
    tail = tail.at[:, : N_EMB - _NW * _SHARD - _CHUNK].set(
        emb[_NW * _SHARD + _CHUNK:].T)
    scr = _scan_gather(x.astype(jnp.int32), emb.T, tail)
    out_t = _transpose_out(scr)
    return out_t.T


# speedup vs baseline: 1.5370x; 1.5370x over previous
import functools

import jax
import jax.numpy as jnp
from jax import lax
from jax.experimental import pallas as pl
from jax.experimental.pallas import tpu as pltpu
from jax.experimental.pallas import tpu_sc as plsc

N_EMB = 1000000
D_EMB = 64
BATCH = 16384

_info = plsc.get_sparse_core_info()
_NC, _NS = _info.num_cores, _info.num_subcores
_NW = _NC * _NS
_BPW = BATCH // _NW
_CHUNK = 128
_NCHUNK = _BPW // _CHUNK

_mesh = plsc.VectorSubcoreMesh(core_axis_name="c", subcore_axis_name="s")


@functools.partial(
    pl.kernel,
    mesh=_mesh,
    out_type=jax.ShapeDtypeStruct((_NW, _NCHUNK, _CHUNK, 128), jnp.float32),
    scratch_types=[
        pltpu.VMEM((_NCHUNK, _CHUNK), jnp.int32),
        pltpu.VMEM((_NCHUNK, _CHUNK, 128), jnp.float32),
        pltpu.SemaphoreType.DMA,
    ],
)
def _emb_lookup(x_hbm, emb_hbm, out_hbm, idx_v, rows_v, sem):
    wid = lax.axis_index("s") * _NC + lax.axis_index("c")
    pltpu.sync_copy(x_hbm.at[wid], idx_v)
    copies = []
    for j in range(_NCHUNK):
        copies.append(
            pltpu.async_copy(emb_hbm.at[idx_v.at[j]], rows_v.at[j], sem))
    for c in copies:
        c.wait()
    pltpu.sync_copy(rows_v, out_hbm.at[wid])


def kernel(x, emb):
    xi = x.astype(jnp.int32)
    phys = (xi >> 1).reshape(_NW, _NCHUNK, _CHUNK)
    emb2 = emb.reshape(N_EMB // 2, 128)
    out = _emb_lookup(phys, emb2).reshape(BATCH, 128)
    return jnp.where((xi & 1)[:, None] == 1, out[:, 64:], out[:, :64])


# speedup vs baseline: 1.8044x; 1.1739x over previous
import functools

import jax
import jax.numpy as jnp
from jax import lax
from jax.experimental import pallas as pl
from jax.experimental.pallas import tpu as pltpu
from jax.experimental.pallas import tpu_sc as plsc

N_EMB = 1000000
D_EMB = 64
BATCH = 16384

_info = plsc.get_sparse_core_info()
_NC, _NS = _info.num_cores, _info.num_subcores
_NW = _NC * _NS
_SHARD = 31232
_CHUNK = 512
_NCHUNK = _SHARD // _CHUNK
_DUMP = BATCH
_SEG = 1152
_REG = _NS * _SEG

_mesh = plsc.VectorSubcoreMesh(core_axis_name="c", subcore_axis_name="s")
_params = pltpu.CompilerParams(needs_layout_passes=False)


@functools.partial(
    pl.kernel,
    mesh=_mesh,
    out_type=(
        jax.ShapeDtypeStruct((2 * _REG, 128), jnp.float32),
        jax.ShapeDtypeStruct((2 * _REG,), jnp.int32),
    ),
    compiler_params=_params,
    scratch_types=[
        pltpu.VMEM((BATCH,), jnp.int32),
        pltpu.VMEM((BATCH + 16,), jnp.int32),
        pltpu.VMEM((BATCH + 16,), jnp.int32),
        pltpu.VMEM((BATCH + 16,), jnp.int32),
        pltpu.VMEM((D_EMB, _CHUNK), jnp.float32),
        pltpu.VMEM((2, 64, 128), jnp.float32),
        pltpu.VMEM((2, 64), jnp.int32),
        pltpu.SMEM((1,), jnp.int32),
        pltpu.SemaphoreType.DMA,
        pltpu.SemaphoreType.DMA,
    ],
)
def _scan_gather(x_hbm, embt_hbm, tail_hbm, rows_hbm, side_hbm,
                 idx_v, mb_v, cb_v, cr_v, stage_v, acc_v, aidx_v,
                 cnt_s, sem, sem_sc):
    sid = lax.axis_index("s")
    cid = lax.axis_index("c")
    wid = sid * _NC + cid
    lo = wid * _SHARD
    hi = jnp.where(wid == _NW - 1, N_EMB, lo + _SHARD).astype(jnp.int32)
    iota = lax.iota(jnp.int32, 16)

    pltpu.sync_copy(x_hbm, idx_v)

    def prefilter(g, off):
        v = idx_v[pl.ds(g * 16, 16)]
        m = (v >= lo) & (v < hi)
        plsc.store_compressed(mb_v.at[pl.ds(off, 16)], g * 16 + iota, mask=m)
        return off + plsc.all_reduce_population_count(m)[0]

    n_match = lax.fori_loop(0, BATCH // 16, prefilter, jnp.int32(0))
    n_groups = (n_match + 15) // 16

    for k16 in range(4):
        aidx_v[0, pl.ds(k16 * 16, 16)] = jnp.full((16,), _DUMP, jnp.int32)
    pre0 = cid * _REG + sid * _SEG
    for k in range(_SEG // 64):
        pltpu.sync_copy(aidx_v.at[0],
                        side_hbm.at[pl.ds(pre0 + k * 64, 64)])

    @pl.when(sid == 0)
    def _():
        cnt_s[0] = 0

    plsc.subcore_barrier()
    seg = plsc.fetch_and_add(cnt_s.at[0], (n_match + 63) // 64 * 64,
                             subcore_id=0)
    base = cid * _REG + seg

    def drain_one():
        pltpu.make_async_copy(
            embt_hbm.at[pl.ds(0, 64), pl.ds(0, 128)], acc_v.at[0],
            sem_sc).wait()
        pltpu.make_async_copy(
            x_hbm.at[pl.ds(0, 64)], aidx_v.at[0], sem_sc).wait()

    def flush(slot, nblk):
        row0 = pl.multiple_of(base + nblk * 64, 8)
        pltpu.async_copy(acc_v.at[slot], rows_hbm.at[pl.ds(row0, 64)],
                         sem_sc)
        pltpu.async_copy(aidx_v.at[slot], side_hbm.at[pl.ds(row0, 64)],
                         sem_sc)

    def process_chunk(rlo, carry):
        rhi = rlo + _CHUNK

        def rescan(g, off):
            mg = (g * 16 + iota) < n_match
            bv = jnp.where(mg, mb_v[pl.ds(g * 16, 16)], 0)
            rv = plsc.load_gather(idx_v, [bv])
            m = mg & (rv >= rlo) & (rv < rhi)
            plsc.store_compressed(cb_v.at[pl.ds(off, 16)], bv, mask=m)
            plsc.store_compressed(cr_v.at[pl.ds(off, 16)], rv, mask=m)
            return off + plsc.all_reduce_population_count(m)[0]

        n2 = lax.fori_loop(0, n_groups, rescan, jnp.int32(0))

        def extract(h, ec):
            fill, slot, o, nblk = ec
            fresh = (fill == 0) & (o >= 2)

            @pl.when(fresh)
            def _():
                drain_one()

            o = jnp.where(fresh, o - 1, o)
            b16 = cb_v[pl.ds(h * 16, 16)]
            r16 = cr_v[pl.ds(h * 16, 16)] - rlo
            mk = (h * 16 + iota) < n2
            r16 = jnp.where(mk, r16, 0)
            slot16 = jnp.full((16,), 0, jnp.int32) + slot
            f16 = fill + iota

            def move_d(dd, _):
                dsplat = jnp.full((16,), 0, jnp.int32) + dd
                val = plsc.load_gather(stage_v, [dsplat, r16], mask=mk)
                plsc.store_scatter(acc_v, [slot16, f16, dsplat], val, mask=mk)
                return 0

            lax.fori_loop(0, D_EMB, move_d, 0)
            plsc.store_scatter(aidx_v, [slot16, f16],
                               jnp.where(mk, b16, _DUMP))
            fill = fill + 16
            fl = fill == 64

            @pl.when(fl)
            def _():
                flush(slot, nblk)

            fli = fl.astype(jnp.int32)
            return (jnp.where(fl, 0, fill), jnp.where(fl, 1 - slot, slot),
                    o + fli, nblk + fli)

        return lax.fori_loop(0, (n2 + 15) // 16, extract, carry)

    def stage_chunk(rlo):
        cps = [
            pltpu.async_copy(
                embt_hbm.at[pl.ds(i * 8, 8),
                            pl.ds(pl.multiple_of(rlo, 128), _CHUNK)],
                stage_v.at[pl.ds(i * 8, 8), pl.ds(0, _CHUNK)],
                sem)
            for i in range(8)
        ]
        for cp in cps:
            cp.wait()

    def chunk_body(c, carry):
        rlo = lo + c * _CHUNK

        @pl.when(c < _NCHUNK + 1)
        def _():
            stage_chunk(rlo)

        @pl.when(c == _NCHUNK + 1)
        def _():
            pltpu.sync_copy(tail_hbm, stage_v.at[:, pl.ds(0, 128)])

        return process_chunk(rlo, carry)

    n_chunks = jnp.where(wid == _NW - 1, _NCHUNK + 2, _NCHUNK)
    fill, slot, o, nblk = lax.fori_loop(
        0, n_chunks, chunk_body,
        (jnp.int32(0), jnp.int32(0), jnp.int32(0), jnp.int32(0)))

    fresh = (fill > 0) & (o >= 2)

    @pl.when(fresh)
    def _():
        drain_one()

    o = jnp.where(fresh, o - 1, o)

    @pl.when(fill > 0)
    def _():
        slot16 = jnp.full((16,), 0, jnp.int32) + slot

        def pad(k, _):
            plsc.store_scatter(aidx_v, [slot16, fill + k * 16 + iota],
                               jnp.full((16,), _DUMP, jnp.int32))
            return 0

        lax.fori_loop(0, (64 - fill) // 16, pad, 0)
        flush(slot, nblk)

    o = o + (fill > 0).astype(jnp.int32)

    def fdrain(k, _):
        drain_one()
        return 0

    lax.fori_loop(0, o, fdrain, 0)


_BPW = BATCH // _NW


@functools.partial(
    pl.kernel,
    mesh=_mesh,
    out_type=jax.ShapeDtypeStruct((D_EMB, BATCH), jnp.float32),
    compiler_params=_params,
    scratch_types=[
        pltpu.VMEM((_REG,), jnp.int32),
        pltpu.VMEM((_BPW + 16,), jnp.int32),
        pltpu.VMEM((_BPW + 16,), jnp.int32),
        pltpu.VMEM((_BPW, 128), jnp.float32),
        pltpu.VMEM((D_EMB, _BPW), jnp.float32),
        pltpu.SemaphoreType.DMA,
    ],
)
def _route_out(rows_hbm, side_hbm, out_hbm, sc_v, pos_v, pb_v, st_v, ob_v,
               sem):
    wid = lax.axis_index("s") * _NC + lax.axis_index("c")
    b0 = wid * _BPW
    iota = lax.iota(jnp.int32, 16)

    def init(g, _):
        pos_v[pl.ds(g * 16, 16)] = jnp.zeros((16,), jnp.int32)
        pb_v[pl.ds(g * 16, 16)] = jnp.full((16,), _DUMP, jnp.int32)
        return 0

    lax.fori_loop(0, _BPW // 16, init, 0)

    off = jnp.int32(0)
    for reg in range(2):
        pltpu.sync_copy(side_hbm.at[pl.ds(reg * _REG, _REG)], sc_v)

        def scan(g, off, reg=reg):
            sv = sc_v[pl.ds(g * 16, 16)]
            m = (sv >= b0) & (sv < b0 + _BPW)
            plsc.store_compressed(pos_v.at[pl.ds(off, 16)],
                                  reg * _REG + g * 16 + iota, mask=m)
            plsc.store_compressed(pb_v.at[pl.ds(off, 16)], sv, mask=m)
            return off + plsc.all_reduce_population_count(m)[0]

        off = lax.fori_loop(0, _REG // 16, scan, off)

    cps = [
        pltpu.async_copy(
            rows_hbm.at[pos_v.at[pl.ds(k * 128, 128)]],
            st_v.at[pl.ds(k * 128, 128)], sem)
        for k in range(_BPW // 128)
    ]
    for cp in cps:
        cp.wait()

    def grp(h, _):
        p16 = h * 16 + iota
        col16 = pb_v[pl.ds(h * 16, 16)] - b0
        mk = (col16 >= 0) & (col16 < _BPW)
        col16 = jnp.where(mk, col16, 0)
        for dd in range(D_EMB):
            dsplat = jnp.full((16,), dd, jnp.int32)
            val = plsc.load_gather(st_v, [p16, dsplat], mask=mk)
            plsc.store_scatter(ob_v, [dsplat, col16], val, mask=mk)
        return 0

    lax.fori_loop(0, _BPW // 16, grp, 0)
    pltpu.sync_copy(ob_v, out_hbm.at[:, pl.ds(pl.multiple_of(b0, 128), _BPW)])


def kernel(x, emb):
    tail = jnp.zeros((D_EMB, 128), jnp.float32)
    tail = tail.at[:, : N_EMB - _NW * _SHARD - _CHUNK].set(
        emb[_NW * _SHARD + _CHUNK:].T)
    rows, side = _scan_gather(x.astype(jnp.int32), emb.T, tail)
    out_t = _route_out(rows, side)
    return out_t.T
